# baseline (speedup 1.0000x reference)
import functools

import jax
import jax.numpy as jnp
from jax import lax
from jax.experimental import pallas as pl
from jax.experimental.pallas import tpu as pltpu
from jax.experimental.pallas import tpu_sc as plsc

N = 10000
E = 320000
D = 128
H = 64
C = 2

NC, NS = 2, 16
NW = NC * NS
CHUNK = 128
NBUF = 3
EPT = -(-E // NW // (CHUNK * NBUF)) * CHUNK * NBUF
EPAD = EPT * NW
CPT = EPT // CHUNK
OUTER = CPT // NBUF
NP_ = 10240
NPT = NP_ // NS
NACC = NP_
DEGW = 16

_mesh = plsc.VectorSubcoreMesh(
    core_axis_name="c", subcore_axis_name="s", num_cores=NC, num_subcores=NS
)
_sc_params = pltpu.CompilerParams(use_tc_tiling_on_sc=False)


def _agg_body(y_hbm, src_hbm, dst_hbm, out_hbm, acc, ysp, srcb, dstb,
              rows_, gsems_, ssems_):
    rows = list(rows_)
    gsems = list(gsems_)
    ssems = list(ssems_)
    c = lax.axis_index("c")
    s = lax.axis_index("s")
    wid = s * NC + c
    pltpu.sync_copy(src_hbm.at[wid], srcb)
    pltpu.sync_copy(dst_hbm.at[wid], dstb)
    pltpu.sync_copy(y_hbm.at[pl.ds(s * NPT, NPT)], acc.at[pl.ds(s * NPT, NPT)])
    pltpu.sync_copy(y_hbm.at[pl.ds(s * NPT, NPT)], ysp.at[pl.ds(s * NPT, NPT)])
    plsc.subcore_barrier()
    for b in range(NBUF):
        ysrc = y_hbm if b == 0 else ysp
        pltpu.async_copy(ysrc.at[srcb.at[b]], rows[b], gsems[b])

    def outer(kk, carry):
        for b in range(NBUF):
            k = kk * NBUF + b
            pltpu.make_async_copy(ysp.at[srcb.at[b]], rows[b], gsems[b]).wait()
            pltpu.async_copy(rows[b], acc.at[dstb.at[k]], ssems[b], add=True)
        for b in range(NBUF):
            k = kk * NBUF + b
            pltpu.make_async_copy(rows[b], acc.at[dstb.at[k]], ssems[b]).wait()

            ysrc = y_hbm if b == 0 else ysp

            @pl.when(kk < OUTER - 1)
            def _():
                pltpu.async_copy(ysrc.at[srcb.at[k + NBUF]], rows[b], gsems[b])

        return carry

    lax.fori_loop(0, OUTER, outer, 0)
    plsc.subcore_barrier()
    pltpu.sync_copy(acc.at[pl.ds(s * NPT, NPT)], out_hbm.at[c, pl.ds(s * NPT, NPT)])


_agg_call = pl.kernel(
    _agg_body,
    out_type=jax.ShapeDtypeStruct((NC, NP_, H), jnp.float32),
    mesh=_mesh,
    scratch_types=[
        pltpu.VMEM_SHARED((NACC, H), jnp.float32),
        pltpu.VMEM_SHARED((NACC, H), jnp.float32),
        pltpu.VMEM((CPT, CHUNK), jnp.int32),
        pltpu.VMEM((CPT, CHUNK), jnp.int32),
        [pltpu.VMEM((CHUNK, H), jnp.float32) for _ in range(NBUF)],
        [pltpu.SemaphoreType.DMA for _ in range(NBUF)],
        [pltpu.SemaphoreType.DMA for _ in range(NBUF)],
    ],
    compiler_params=_sc_params,
)


def _deg_body(ones_hbm, zeros_hbm, dst_hbm, out_hbm, acc, ones_v, dstb, sem):
    c = lax.axis_index("c")
    s = lax.axis_index("s")
    wid = s * NC + c
    pltpu.sync_copy(dst_hbm.at[wid], dstb)
    pltpu.sync_copy(zeros_hbm.at[pl.ds(s * NPT, NPT)], acc.at[pl.ds(s * NPT, NPT)])
    pltpu.sync_copy(ones_hbm, ones_v)
    plsc.subcore_barrier()

    def fire(k, carry):
        pltpu.async_copy(ones_v, acc.at[dstb.at[k]], sem, add=True)
        return carry

    lax.fori_loop(0, CPT, fire, 0)

    def drain(k, carry):
        pltpu.make_async_copy(ones_v, acc.at[dstb.at[0]], sem).wait()
        return carry

    lax.fori_loop(0, CPT, drain, 0)
    plsc.subcore_barrier()
    pltpu.sync_copy(acc.at[pl.ds(s * NPT, NPT)], out_hbm.at[c, pl.ds(s * NPT, NPT)])


_deg_call = pl.kernel(
    _deg_body,
    out_type=jax.ShapeDtypeStruct((NC, NP_, DEGW), jnp.float32),
    mesh=_mesh,
    scratch_types=[
        pltpu.VMEM_SHARED((NACC, DEGW), jnp.float32),
        pltpu.VMEM((CHUNK, DEGW), jnp.float32),
        pltpu.VMEM((CPT, CHUNK), jnp.int32),
        pltpu.SemaphoreType.DMA,
    ],
    compiler_params=_sc_params,
)


_RB = 1024


def _mm_body(x_ref, w_ref, o_ref):
    o_ref[...] = jnp.dot(x_ref[...], w_ref[...], preferred_element_type=jnp.float32)


def _mm(x, w):
    n, d = x.shape
    h = w.shape[1]
    return pl.pallas_call(
        _mm_body,
        grid=(n // _RB,),
        in_specs=[
            pl.BlockSpec((_RB, d), lambda i: (i, 0)),
            pl.BlockSpec((d, h), lambda i: (0, 0)),
        ],
        out_specs=pl.BlockSpec((_RB, h), lambda i: (i, 0)),
        out_shape=jax.ShapeDtypeStruct((n, h), jnp.float32),
    )(x, w)


def _norm_body(degp_ref, u_ref, y_ref, dinv_ref):
    deg = degp_ref[0][:, 0:1] + degp_ref[1][:, 0:1] + 1.0
    dinv = lax.rsqrt(deg)
    y_ref[...] = dinv * u_ref[...]
    dinv_ref[...] = dinv


def _norm(degp, u):
    return pl.pallas_call(
        _norm_body,
        grid=(NP_ // _RB,),
        in_specs=[
            pl.BlockSpec((NC, _RB, DEGW), lambda i: (0, i, 0)),
            pl.BlockSpec((_RB, H), lambda i: (i, 0)),
        ],
        out_specs=[
            pl.BlockSpec((_RB, H), lambda i: (i, 0)),
            pl.BlockSpec((_RB, 1), lambda i: (i, 0)),
        ],
        out_shape=[
            jax.ShapeDtypeStruct((NP_, H), jnp.float32),
            jax.ShapeDtypeStruct((NP_, 1), jnp.float32),
        ],
    )(degp, u)


def _next_body(p_ref, yprev_ref, dinv_ref, b_ref, w_ref, y_ref):
    agg = p_ref[0] + p_ref[1] - yprev_ref[...]
    h = jnp.maximum(dinv_ref[...] * agg + b_ref[...], 0.0)
    y_ref[...] = dinv_ref[...] * jnp.dot(
        h, w_ref[...], preferred_element_type=jnp.float32
    )


def _next_layer(p, yprev, dinv, b, w):
    return pl.pallas_call(
        _next_body,
        grid=(NP_ // _RB,),
        in_specs=[
            pl.BlockSpec((NC, _RB, H), lambda i: (0, i, 0)),
            pl.BlockSpec((_RB, H), lambda i: (i, 0)),
            pl.BlockSpec((_RB, 1), lambda i: (i, 0)),
            pl.BlockSpec((1, H), lambda i: (0, 0)),
            pl.BlockSpec((H, H), lambda i: (0, 0)),
        ],
        out_specs=pl.BlockSpec((_RB, H), lambda i: (i, 0)),
        out_shape=jax.ShapeDtypeStruct((NP_, H), jnp.float32),
    )(p, yprev, dinv, b, w)


def _final_body(p_ref, yprev_ref, dinv_ref, b_ref, wlin_ref, blin_ref, o_ref):
    agg = p_ref[0] + p_ref[1] - yprev_ref[...]
    h = jnp.maximum(dinv_ref[...] * agg + b_ref[...], 0.0)
    logits = jnp.dot(h, wlin_ref[...], preferred_element_type=jnp.float32)
    logits = logits + blin_ref[...]
    m = jnp.max(logits, axis=1, keepdims=True)
    lse = m + jnp.log(jnp.sum(jnp.exp(logits - m), axis=1, keepdims=True))
    o_ref[...] = logits - lse


def _final(p, yprev, dinv, b, wlin, blin):
    return pl.pallas_call(
        _final_body,
        grid=(NP_ // _RB,),
        in_specs=[
            pl.BlockSpec((NC, _RB, H), lambda i: (0, i, 0)),
            pl.BlockSpec((_RB, H), lambda i: (i, 0)),
            pl.BlockSpec((_RB, 1), lambda i: (i, 0)),
            pl.BlockSpec((1, H), lambda i: (0, 0)),
            pl.BlockSpec((H, C), lambda i: (0, 0)),
            pl.BlockSpec((1, C), lambda i: (0, 0)),
        ],
        out_specs=pl.BlockSpec((_RB, C), lambda i: (i, 0)),
        out_shape=jax.ShapeDtypeStruct((NP_, C), jnp.float32),
    )(p, yprev, dinv, b, wlin, blin)


@jax.jit
def kernel(x, edge_index, W1, b1, W2, b2, W3, b3, Wlin, blin):
    src = edge_index[0].astype(jnp.int32)
    dst = edge_index[1].astype(jnp.int32)
    pad = EPAD - E
    srcp = jnp.concatenate([src, jnp.zeros((pad,), jnp.int32)]).reshape(NW, CPT, CHUNK)
    dstp = jnp.concatenate([dst, jnp.full((pad,), N, jnp.int32)]).reshape(NW, CPT, CHUNK)

    ones = jnp.ones((CHUNK, DEGW), jnp.float32)
    zeros = jnp.zeros((NP_, DEGW), jnp.float32)
    xp = jnp.pad(x, ((0, NP_ - N), (0, 0)))

    degp = _deg_call(ones, zeros, dstp)
    u1 = _mm(xp, W1)
    y1, dinv = _norm(degp, u1)

    p1 = _agg_call(y1, srcp, dstp)
    y2 = _next_layer(p1, y1, dinv, b1.reshape(1, H), W2)
    p2 = _agg_call(y2, srcp, dstp)
    y3 = _next_layer(p2, y2, dinv, b2.reshape(1, H), W3)
    p3 = _agg_call(y3, srcp, dstp)
    out = _final(p3, y3, dinv, b3.reshape(1, H), Wlin, blin.reshape(1, C))
    return out[:N]


# speedup vs baseline: 1.4711x; 1.4711x over previous
import functools

import jax
import jax.numpy as jnp
from jax import lax
from jax.experimental import pallas as pl
from jax.experimental.pallas import tpu as pltpu
from jax.experimental.pallas import tpu_sc as plsc

N = 10000
E = 320000
D = 128
H = 64
C = 2

NC, NS = 2, 16
NW = NC * NS
CHUNK = 128
NBUF = 3
EPT = -(-E // NW // (CHUNK * NBUF)) * CHUNK * NBUF
EPAD = EPT * NW
CPT = EPT // CHUNK
OUTER = CPT // NBUF
NP_ = 10240
NPT = NP_ // NS
NACC = NP_
DEGW = 16

_mesh = plsc.VectorSubcoreMesh(
    core_axis_name="c", subcore_axis_name="s", num_cores=NC, num_subcores=NS
)
_sc_params = pltpu.CompilerParams(use_tc_tiling_on_sc=False)


def _agg_body(y_hbm, src_hbm, dst_hbm, out_hbm, acc, ysp, srcb, dstb,
              rows_, gsems_, ssems_):
    rows = list(rows_)
    gsems = list(gsems_)
    ssems = list(ssems_)
    c = lax.axis_index("c")
    s = lax.axis_index("s")
    wid = s * NC + c
    pltpu.sync_copy(src_hbm.at[wid], srcb)
    pltpu.sync_copy(dst_hbm.at[wid], dstb)
    pltpu.sync_copy(y_hbm.at[pl.ds(s * NPT, NPT)], acc.at[pl.ds(s * NPT, NPT)])
    pltpu.sync_copy(y_hbm.at[pl.ds(s * NPT, NPT)], ysp.at[pl.ds(s * NPT, NPT)])
    plsc.subcore_barrier()
    for b in range(NBUF):
        pltpu.async_copy(ysp.at[srcb.at[b]], rows[b], gsems[b])

    def outer(kk, carry):
        for b in range(NBUF):
            k = kk * NBUF + b
            pltpu.make_async_copy(ysp.at[srcb.at[b]], rows[b], gsems[b]).wait()
            pltpu.async_copy(rows[b], acc.at[dstb.at[k]], ssems[b], add=True)
        for b in range(NBUF):
            k = kk * NBUF + b
            pltpu.make_async_copy(rows[b], acc.at[dstb.at[k]], ssems[b]).wait()

            @pl.when(kk < OUTER - 1)
            def _():
                pltpu.async_copy(ysp.at[srcb.at[k + NBUF]], rows[b], gsems[b])

        return carry

    lax.fori_loop(0, OUTER, outer, 0)
    plsc.subcore_barrier()
    pltpu.sync_copy(acc.at[pl.ds(s * NPT, NPT)], out_hbm.at[c, pl.ds(s * NPT, NPT)])


_agg_call = pl.kernel(
    _agg_body,
    out_type=jax.ShapeDtypeStruct((NC, NP_, H), jnp.float32),
    mesh=_mesh,
    scratch_types=[
        pltpu.VMEM_SHARED((NACC, H), jnp.float32),
        pltpu.VMEM_SHARED((NACC, H), jnp.float32),
        pltpu.VMEM((CPT, CHUNK), jnp.int32),
        pltpu.VMEM((CPT, CHUNK), jnp.int32),
        [pltpu.VMEM((CHUNK, H), jnp.float32) for _ in range(NBUF)],
        [pltpu.SemaphoreType.DMA for _ in range(NBUF)],
        [pltpu.SemaphoreType.DMA for _ in range(NBUF)],
    ],
    compiler_params=_sc_params,
)


def _deg_body(ones_hbm, zeros_hbm, dst_hbm, out_hbm, acc, ones_v, dstb, sem):
    c = lax.axis_index("c")
    s = lax.axis_index("s")
    wid = s * NC + c
    pltpu.sync_copy(dst_hbm.at[wid], dstb)
    pltpu.sync_copy(zeros_hbm.at[pl.ds(s * NPT, NPT)], acc.at[pl.ds(s * NPT, NPT)])
    pltpu.sync_copy(ones_hbm, ones_v)
    plsc.subcore_barrier()

    def fire(k, carry):
        pltpu.async_copy(ones_v, acc.at[dstb.at[k]], sem, add=True)
        return carry

    lax.fori_loop(0, CPT, fire, 0)

    def drain(k, carry):
        pltpu.make_async_copy(ones_v, acc.at[dstb.at[0]], sem).wait()
        return carry

    lax.fori_loop(0, CPT, drain, 0)
    plsc.subcore_barrier()
    pltpu.sync_copy(acc.at[pl.ds(s * NPT, NPT)], out_hbm.at[c, pl.ds(s * NPT, NPT)])


_deg_call = pl.kernel(
    _deg_body,
    out_type=jax.ShapeDtypeStruct((NC, NP_, DEGW), jnp.float32),
    mesh=_mesh,
    scratch_types=[
        pltpu.VMEM_SHARED((NACC, DEGW), jnp.float32),
        pltpu.VMEM((CHUNK, DEGW), jnp.float32),
        pltpu.VMEM((CPT, CHUNK), jnp.int32),
        pltpu.SemaphoreType.DMA,
    ],
    compiler_params=_sc_params,
)


_RB = 1024
NPK = NP_ // 2


def _mm_body(x_ref, w_ref, o_ref):
    o_ref[...] = jnp.dot(x_ref[...], w_ref[...], preferred_element_type=jnp.float32)


def _mm(x, w):
    n, d = x.shape
    h = w.shape[1]
    return pl.pallas_call(
        _mm_body,
        grid=(n // _RB,),
        in_specs=[
            pl.BlockSpec((_RB, d), lambda i: (i, 0)),
            pl.BlockSpec((d, h), lambda i: (0, 0)),
        ],
        out_specs=pl.BlockSpec((_RB, h), lambda i: (i, 0)),
        out_shape=jax.ShapeDtypeStruct((n, h), jnp.float32),
    )(x, w)


def _dinv16_body(degp_ref, o_ref):
    deg = degp_ref[0] + degp_ref[1] + 1.0
    o_ref[...] = lax.rsqrt(deg)


def _dinv16(degpv):
    r = degpv.shape[1]
    return pl.pallas_call(
        _dinv16_body,
        grid=(r // 640,),
        in_specs=[pl.BlockSpec((NC, 640, 128), lambda i: (0, i, 0))],
        out_specs=pl.BlockSpec((640, 128), lambda i: (i, 0)),
        out_shape=jax.ShapeDtypeStruct((r, 128), jnp.float32),
    )(degpv)


def _scale_body(d16_ref, t_ref, u_ref, y_ref, dinv_ref):
    dinvp = jnp.dot(d16_ref[...], t_ref[...], preferred_element_type=jnp.float32)
    y_ref[...] = dinvp * u_ref[...]
    dinv_ref[...] = dinvp


def _scale(d16r, t, u):
    return pl.pallas_call(
        _scale_body,
        grid=(NPK // _RB,),
        in_specs=[
            pl.BlockSpec((_RB, 32), lambda i: (i, 0)),
            pl.BlockSpec((32, 128), lambda i: (0, 0)),
            pl.BlockSpec((_RB, 128), lambda i: (i, 0)),
        ],
        out_specs=[
            pl.BlockSpec((_RB, 128), lambda i: (i, 0)),
            pl.BlockSpec((_RB, 128), lambda i: (i, 0)),
        ],
        out_shape=[
            jax.ShapeDtypeStruct((NPK, 128), jnp.float32),
            jax.ShapeDtypeStruct((NPK, 128), jnp.float32),
        ],
    )(d16r, t, u)


def _next_body(p_ref, yprev_ref, dinv_ref, b_ref, w_ref, y_ref):
    agg = p_ref[0] + p_ref[1] - yprev_ref[...]
    h = jnp.maximum(dinv_ref[...] * agg + b_ref[...], 0.0)
    y_ref[...] = dinv_ref[...] * jnp.dot(
        h, w_ref[...], preferred_element_type=jnp.float32
    )


def _next_layer(pv, yprev, dinv, b2, w2):
    return pl.pallas_call(
        _next_body,
        grid=(NPK // _RB,),
        in_specs=[
            pl.BlockSpec((NC, _RB, 128), lambda i: (0, i, 0)),
            pl.BlockSpec((_RB, 128), lambda i: (i, 0)),
            pl.BlockSpec((_RB, 128), lambda i: (i, 0)),
            pl.BlockSpec((1, 128), lambda i: (0, 0)),
            pl.BlockSpec((128, 128), lambda i: (0, 0)),
        ],
        out_specs=pl.BlockSpec((_RB, 128), lambda i: (i, 0)),
        out_shape=jax.ShapeDtypeStruct((NPK, 128), jnp.float32),
    )(pv, yprev, dinv, b2, w2)


def _final_body(p_ref, yprev_ref, dinv_ref, b_ref, wlin_ref, blin_ref, o_ref):
    agg = p_ref[0] + p_ref[1] - yprev_ref[...]
    h = jnp.maximum(dinv_ref[...] * agg + b_ref[...], 0.0)
    lg = jnp.dot(h, wlin_ref[...], preferred_element_type=jnp.float32)
    lg = lg + blin_ref[...]

    def lsm(l):
        m = jnp.max(l, axis=1, keepdims=True)
        return l - (m + jnp.log(jnp.sum(jnp.exp(l - m), axis=1, keepdims=True)))

    o_ref[...] = jnp.concatenate([lsm(lg[:, 0:2]), lsm(lg[:, 2:4])], axis=1)


def _final(pv, yprev, dinv, b2, wlin2, blin2):
    return pl.pallas_call(
        _final_body,
        grid=(NPK // _RB,),
        in_specs=[
            pl.BlockSpec((NC, _RB, 128), lambda i: (0, i, 0)),
            pl.BlockSpec((_RB, 128), lambda i: (i, 0)),
            pl.BlockSpec((_RB, 128), lambda i: (i, 0)),
            pl.BlockSpec((1, 128), lambda i: (0, 0)),
            pl.BlockSpec((128, 4), lambda i: (0, 0)),
            pl.BlockSpec((1, 4), lambda i: (0, 0)),
        ],
        out_specs=pl.BlockSpec((_RB, 4), lambda i: (i, 0)),
        out_shape=jax.ShapeDtypeStruct((NPK, 4), jnp.float32),
    )(pv, yprev, dinv, b2, wlin2, blin2)


def _blockdiag(w):
    d, h = w.shape
    z1 = jnp.zeros((d, h), jnp.float32)
    top = jnp.concatenate([w, z1], axis=1)
    bot = jnp.concatenate([z1, w], axis=1)
    return jnp.concatenate([top, bot], axis=0)


@jax.jit
def kernel(x, edge_index, W1, b1, W2, b2, W3, b3, Wlin, blin):
    src = edge_index[0].astype(jnp.int32)
    dst = edge_index[1].astype(jnp.int32)
    pad = EPAD - E
    srcp = jnp.concatenate([src, jnp.zeros((pad,), jnp.int32)]).reshape(NW, CPT, CHUNK)
    dstp = jnp.concatenate([dst, jnp.full((pad,), N, jnp.int32)]).reshape(NW, CPT, CHUNK)

    ones = jnp.ones((CHUNK, DEGW), jnp.float32)
    zeros = jnp.zeros((NP_, DEGW), jnp.float32)
    xp = jnp.pad(x, ((0, NP_ - N), (0, 0)))

    w1b = _blockdiag(W1)
    w2b = _blockdiag(W2)
    w3b = _blockdiag(W3)
    wlb = _blockdiag(Wlin)
    b1x = jnp.concatenate([b1, b1]).reshape(1, 2 * H)
    b2x = jnp.concatenate([b2, b2]).reshape(1, 2 * H)
    b3x = jnp.concatenate([b3, b3]).reshape(1, 2 * H)
    blx = jnp.concatenate([blin, blin]).reshape(1, 2 * C)
    t = jnp.zeros((32, 128), jnp.float32)
    t = t.at[0, 0:64].set(1.0).at[16, 64:128].set(1.0)

    degp = _deg_call(ones, zeros, dstp)
    u1 = _mm(xp.reshape(NPK, 2 * D), w1b)
    d16 = _dinv16(degp.reshape(NC, NP_ * DEGW // 128, 128))
    y1, dinv = _scale(d16.reshape(NPK, 32), t, u1)

    p1 = _agg_call(y1.reshape(NP_, H), srcp, dstp)
    y2 = _next_layer(p1.reshape(NC, NPK, 128), y1, dinv, b1x, w2b)
    p2 = _agg_call(y2.reshape(NP_, H), srcp, dstp)
    y3 = _next_layer(p2.reshape(NC, NPK, 128), y2, dinv, b2x, w3b)
    p3 = _agg_call(y3.reshape(NP_, H), srcp, dstp)
    out = _final(p3.reshape(NC, NPK, 128), y3, dinv, b3x, wlb, blx)
    return out.reshape(NP_, C)[:N]


# speedup vs baseline: 1.5454x; 1.0505x over previous
import functools

import jax
import jax.numpy as jnp
from jax import lax
from jax.experimental import pallas as pl
from jax.experimental.pallas import tpu as pltpu
from jax.experimental.pallas import tpu_sc as plsc

N = 10000
E = 320000
D = 128
H = 64
C = 2

NC, NS = 2, 16
NW = NC * NS
CHUNK = 125
NBUF = 3
EPT = E // NW
CPT = EPT // CHUNK
OUTER = CPT // NBUF
TAIL = CPT - OUTER * NBUF
NP_ = 10240
NPT = NP_ // NS
NACC = NP_
DEGW = 16

_mesh = plsc.VectorSubcoreMesh(
    core_axis_name="c", subcore_axis_name="s", num_cores=NC, num_subcores=NS
)
_sc_params = pltpu.CompilerParams(use_tc_tiling_on_sc=False)


def _agg_body(y_hbm, src_hbm, dst_hbm, out_hbm, acc, ysp, srcb, dstb,
              rows_, gsems_, ssems_):
    rows = list(rows_)
    gsems = list(gsems_)
    ssems = list(ssems_)
    c = lax.axis_index("c")
    s = lax.axis_index("s")
    wid = s * NC + c
    pltpu.sync_copy(src_hbm.at[wid], srcb)
    pltpu.sync_copy(dst_hbm.at[wid], dstb)
    pltpu.sync_copy(y_hbm.at[pl.ds(s * NPT, NPT)], acc.at[pl.ds(s * NPT, NPT)])
    pltpu.sync_copy(y_hbm.at[pl.ds(s * NPT, NPT)], ysp.at[pl.ds(s * NPT, NPT)])
    plsc.subcore_barrier()
    for b in range(NBUF):
        pltpu.async_copy(ysp.at[srcb.at[b]], rows[b], gsems[b])

    def outer(kk, carry):
        for b in range(NBUF):
            k = kk * NBUF + b
            pltpu.make_async_copy(ysp.at[srcb.at[b]], rows[b], gsems[b]).wait()
            pltpu.async_copy(rows[b], acc.at[dstb.at[k]], ssems[b], add=True)
        for b in range(NBUF):
            k = kk * NBUF + b
            pltpu.make_async_copy(rows[b], acc.at[dstb.at[k]], ssems[b]).wait()

            @pl.when(kk < OUTER - 1)
            def _():
                pltpu.async_copy(ysp.at[srcb.at[k + NBUF]], rows[b], gsems[b])

        return carry

    lax.fori_loop(0, OUTER, outer, 0)
    for i in range(TAIL):
        k = OUTER * NBUF + i
        pltpu.async_copy(ysp.at[srcb.at[k]], rows[i], gsems[i])
    for i in range(TAIL):
        k = OUTER * NBUF + i
        pltpu.make_async_copy(ysp.at[srcb.at[k]], rows[i], gsems[i]).wait()
        pltpu.sync_copy(rows[i], acc.at[dstb.at[k]], add=True)
    plsc.subcore_barrier()
    pltpu.sync_copy(acc.at[pl.ds(s * NPT, NPT)], out_hbm.at[c, pl.ds(s * NPT, NPT)])


_agg_call = pl.kernel(
    _agg_body,
    out_type=jax.ShapeDtypeStruct((NC, NP_, H), jnp.float32),
    mesh=_mesh,
    scratch_types=[
        pltpu.VMEM_SHARED((NACC, H), jnp.float32),
        pltpu.VMEM_SHARED((NACC, H), jnp.float32),
        pltpu.VMEM((CPT, CHUNK), jnp.int32),
        pltpu.VMEM((CPT, CHUNK), jnp.int32),
        [pltpu.VMEM((CHUNK, H), jnp.float32) for _ in range(NBUF)],
        [pltpu.SemaphoreType.DMA for _ in range(NBUF)],
        [pltpu.SemaphoreType.DMA for _ in range(NBUF)],
    ],
    compiler_params=_sc_params,
)


def _deg_body(ones_hbm, zeros_hbm, dst_hbm, out_hbm, acc, ones_v, dstb, sem):
    c = lax.axis_index("c")
    s = lax.axis_index("s")
    wid = s * NC + c
    pltpu.sync_copy(dst_hbm.at[wid], dstb)
    pltpu.sync_copy(zeros_hbm.at[pl.ds(s * NPT, NPT)], acc.at[pl.ds(s * NPT, NPT)])
    pltpu.sync_copy(ones_hbm, ones_v)
    plsc.subcore_barrier()

    def fire(k, carry):
        pltpu.async_copy(ones_v, acc.at[dstb.at[k]], sem, add=True)
        return carry

    lax.fori_loop(0, CPT, fire, 0)

    def drain(k, carry):
        pltpu.make_async_copy(ones_v, acc.at[dstb.at[0]], sem).wait()
        return carry

    lax.fori_loop(0, CPT, drain, 0)
    plsc.subcore_barrier()
    pltpu.sync_copy(acc.at[pl.ds(s * NPT, NPT)], out_hbm.at[c, pl.ds(s * NPT, NPT)])


_deg_call = pl.kernel(
    _deg_body,
    out_type=jax.ShapeDtypeStruct((NC, NP_, DEGW), jnp.float32),
    mesh=_mesh,
    scratch_types=[
        pltpu.VMEM_SHARED((NACC, DEGW), jnp.float32),
        pltpu.VMEM((CHUNK, DEGW), jnp.float32),
        pltpu.VMEM((CPT, CHUNK), jnp.int32),
        pltpu.SemaphoreType.DMA,
    ],
    compiler_params=_sc_params,
)


_RB = 1024
NPK = NP_ // 2


def _mm_body(x_ref, w_ref, o_ref):
    o_ref[...] = jnp.dot(x_ref[...], w_ref[...], preferred_element_type=jnp.float32)


def _mm(x, w):
    n, d = x.shape
    h = w.shape[1]
    return pl.pallas_call(
        _mm_body,
        grid=(n // _RB,),
        in_specs=[
            pl.BlockSpec((_RB, d), lambda i: (i, 0)),
            pl.BlockSpec((d, h), lambda i: (0, 0)),
        ],
        out_specs=pl.BlockSpec((_RB, h), lambda i: (i, 0)),
        out_shape=jax.ShapeDtypeStruct((n, h), jnp.float32),
    )(x, w)


def _dinv16_body(degp_ref, o_ref):
    deg = degp_ref[0] + degp_ref[1] + 1.0
    o_ref[...] = lax.rsqrt(deg)


def _dinv16(degpv):
    r = degpv.shape[1]
    return pl.pallas_call(
        _dinv16_body,
        grid=(r // 640,),
        in_specs=[pl.BlockSpec((NC, 640, 128), lambda i: (0, i, 0))],
        out_specs=pl.BlockSpec((640, 128), lambda i: (i, 0)),
        out_shape=jax.ShapeDtypeStruct((r, 128), jnp.float32),
    )(degpv)


def _scale_body(d16_ref, t_ref, u_ref, y_ref, dinv_ref):
    dinvp = jnp.dot(d16_ref[...], t_ref[...], preferred_element_type=jnp.float32)
    y_ref[...] = dinvp * u_ref[...]
    dinv_ref[...] = dinvp


def _scale(d16r, t, u):
    return pl.pallas_call(
        _scale_body,
        grid=(NPK // _RB,),
        in_specs=[
            pl.BlockSpec((_RB, 32), lambda i: (i, 0)),
            pl.BlockSpec((32, 128), lambda i: (0, 0)),
            pl.BlockSpec((_RB, 128), lambda i: (i, 0)),
        ],
        out_specs=[
            pl.BlockSpec((_RB, 128), lambda i: (i, 0)),
            pl.BlockSpec((_RB, 128), lambda i: (i, 0)),
        ],
        out_shape=[
            jax.ShapeDtypeStruct((NPK, 128), jnp.float32),
            jax.ShapeDtypeStruct((NPK, 128), jnp.float32),
        ],
    )(d16r, t, u)


def _next_body(p_ref, yprev_ref, dinv_ref, b_ref, w_ref, y_ref):
    agg = p_ref[0] + p_ref[1] - yprev_ref[...]
    h = jnp.maximum(dinv_ref[...] * agg + b_ref[...], 0.0)
    y_ref[...] = dinv_ref[...] * jnp.dot(
        h, w_ref[...], preferred_element_type=jnp.float32
    )


def _next_layer(pv, yprev, dinv, b2, w2):
    return pl.pallas_call(
        _next_body,
        grid=(NPK // _RB,),
        in_specs=[
            pl.BlockSpec((NC, _RB, 128), lambda i: (0, i, 0)),
            pl.BlockSpec((_RB, 128), lambda i: (i, 0)),
            pl.BlockSpec((_RB, 128), lambda i: (i, 0)),
            pl.BlockSpec((1, 128), lambda i: (0, 0)),
            pl.BlockSpec((128, 128), lambda i: (0, 0)),
        ],
        out_specs=pl.BlockSpec((_RB, 128), lambda i: (i, 0)),
        out_shape=jax.ShapeDtypeStruct((NPK, 128), jnp.float32),
    )(pv, yprev, dinv, b2, w2)


def _final_body(p_ref, yprev_ref, dinv_ref, b_ref, wlin_ref, blin_ref, o_ref):
    agg = p_ref[0] + p_ref[1] - yprev_ref[...]
    h = jnp.maximum(dinv_ref[...] * agg + b_ref[...], 0.0)
    lg = jnp.dot(h, wlin_ref[...], preferred_element_type=jnp.float32)
    lg = lg + blin_ref[...]

    def lsm(l):
        m = jnp.max(l, axis=1, keepdims=True)
        return l - (m + jnp.log(jnp.sum(jnp.exp(l - m), axis=1, keepdims=True)))

    o_ref[...] = jnp.concatenate([lsm(lg[:, 0:2]), lsm(lg[:, 2:4])], axis=1)


def _final(pv, yprev, dinv, b2, wlin2, blin2):
    return pl.pallas_call(
        _final_body,
        grid=(NPK // _RB,),
        in_specs=[
            pl.BlockSpec((NC, _RB, 128), lambda i: (0, i, 0)),
            pl.BlockSpec((_RB, 128), lambda i: (i, 0)),
            pl.BlockSpec((_RB, 128), lambda i: (i, 0)),
            pl.BlockSpec((1, 128), lambda i: (0, 0)),
            pl.BlockSpec((128, 4), lambda i: (0, 0)),
            pl.BlockSpec((1, 4), lambda i: (0, 0)),
        ],
        out_specs=pl.BlockSpec((_RB, 4), lambda i: (i, 0)),
        out_shape=jax.ShapeDtypeStruct((NPK, 4), jnp.float32),
    )(pv, yprev, dinv, b2, wlin2, blin2)


def _blockdiag(w):
    d, h = w.shape
    z1 = jnp.zeros((d, h), jnp.float32)
    top = jnp.concatenate([w, z1], axis=1)
    bot = jnp.concatenate([z1, w], axis=1)
    return jnp.concatenate([top, bot], axis=0)


@jax.jit
def kernel(x, edge_index, W1, b1, W2, b2, W3, b3, Wlin, blin):
    srcp = edge_index[0].astype(jnp.int32).reshape(NW, CPT, CHUNK)
    dstp = edge_index[1].astype(jnp.int32).reshape(NW, CPT, CHUNK)

    ones = jnp.ones((CHUNK, DEGW), jnp.float32)
    zeros = jnp.zeros((NP_, DEGW), jnp.float32)
    xp = jnp.pad(x, ((0, NP_ - N), (0, 0)))

    w1b = _blockdiag(W1)
    w2b = _blockdiag(W2)
    w3b = _blockdiag(W3)
    wlb = _blockdiag(Wlin)
    b1x = jnp.concatenate([b1, b1]).reshape(1, 2 * H)
    b2x = jnp.concatenate([b2, b2]).reshape(1, 2 * H)
    b3x = jnp.concatenate([b3, b3]).reshape(1, 2 * H)
    blx = jnp.concatenate([blin, blin]).reshape(1, 2 * C)
    t = jnp.zeros((32, 128), jnp.float32)
    t = t.at[0, 0:64].set(1.0).at[16, 64:128].set(1.0)

    degp = _deg_call(ones, zeros, dstp)
    u1 = _mm(xp.reshape(NPK, 2 * D), w1b)
    d16 = _dinv16(degp.reshape(NC, NP_ * DEGW // 128, 128))
    y1, dinv = _scale(d16.reshape(NPK, 32), t, u1)

    p1 = _agg_call(y1.reshape(NP_, H), srcp, dstp)
    y2 = _next_layer(p1.reshape(NC, NPK, 128), y1, dinv, b1x, w2b)
    p2 = _agg_call(y2.reshape(NP_, H), srcp, dstp)
    y3 = _next_layer(p2.reshape(NC, NPK, 128), y2, dinv, b2x, w3b)
    p3 = _agg_call(y3.reshape(NP_, H), srcp, dstp)
    out = _final(p3.reshape(NC, NPK, 128), y3, dinv, b3x, wlb, blx)
    return out.reshape(NP_, C)[:N]


# speedup vs baseline: 1.5852x; 1.0258x over previous
import functools

import jax
import jax.numpy as jnp
from jax import lax
from jax.experimental import pallas as pl
from jax.experimental.pallas import tpu as pltpu
from jax.experimental.pallas import tpu_sc as plsc

N = 10000
E = 320000
D = 128
H = 64
C = 2

NC, NS = 2, 16
NW = NC * NS
CHUNK = 125
NBUF = 3
EPT = E // NW
CPT = EPT // CHUNK
OUTER = CPT // NBUF
TAIL = CPT - OUTER * NBUF
NP_ = 10240
NPT = NP_ // NS
NACC = NP_
DEGW = 16

_mesh = plsc.VectorSubcoreMesh(
    core_axis_name="c", subcore_axis_name="s", num_cores=NC, num_subcores=NS
)
_sc_params = pltpu.CompilerParams(use_tc_tiling_on_sc=False)


def _agg_body(y_hbm, src_hbm, dst_hbm, out_hbm, acc, ysp, srcb, dstb,
              rows_, gsems_, ssems_):
    rows = list(rows_)
    gsems = list(gsems_)
    ssems = list(ssems_)
    c = lax.axis_index("c")
    s = lax.axis_index("s")
    wid = s * NC + c
    pltpu.async_copy(src_hbm.at[wid], srcb, gsems[0])
    pltpu.async_copy(dst_hbm.at[wid], dstb, gsems[1])
    pltpu.async_copy(y_hbm.at[pl.ds(s * NPT, NPT)], acc.at[pl.ds(s * NPT, NPT)], gsems[2])
    pltpu.async_copy(y_hbm.at[pl.ds(s * NPT, NPT)], ysp.at[pl.ds(s * NPT, NPT)], ssems[0])
    pltpu.make_async_copy(src_hbm.at[wid], srcb, gsems[0]).wait()
    pltpu.make_async_copy(dst_hbm.at[wid], dstb, gsems[1]).wait()
    pltpu.make_async_copy(y_hbm.at[pl.ds(s * NPT, NPT)], acc.at[pl.ds(s * NPT, NPT)], gsems[2]).wait()
    pltpu.make_async_copy(y_hbm.at[pl.ds(s * NPT, NPT)], ysp.at[pl.ds(s * NPT, NPT)], ssems[0]).wait()
    plsc.subcore_barrier()
    for b in range(NBUF):
        pltpu.async_copy(ysp.at[srcb.at[b]], rows[b], gsems[b])

    def outer(kk, carry):
        for b in range(NBUF):
            k = kk * NBUF + b
            pltpu.make_async_copy(ysp.at[srcb.at[b]], rows[b], gsems[b]).wait()
            pltpu.async_copy(rows[b], acc.at[dstb.at[k]], ssems[b], add=True)
        for b in range(NBUF):
            k = kk * NBUF + b
            pltpu.make_async_copy(rows[b], acc.at[dstb.at[k]], ssems[b]).wait()

            @pl.when(kk < OUTER - 1)
            def _():
                pltpu.async_copy(ysp.at[srcb.at[k + NBUF]], rows[b], gsems[b])

        return carry

    lax.fori_loop(0, OUTER, outer, 0)
    for i in range(TAIL):
        k = OUTER * NBUF + i
        pltpu.async_copy(ysp.at[srcb.at[k]], rows[i], gsems[i])
    for i in range(TAIL):
        k = OUTER * NBUF + i
        pltpu.make_async_copy(ysp.at[srcb.at[k]], rows[i], gsems[i]).wait()
        pltpu.sync_copy(rows[i], acc.at[dstb.at[k]], add=True)
    plsc.subcore_barrier()
    pltpu.sync_copy(acc.at[pl.ds(s * NPT, NPT)], out_hbm.at[c, pl.ds(s * NPT, NPT)])


_agg_call = pl.kernel(
    _agg_body,
    out_type=jax.ShapeDtypeStruct((NC, NP_, H), jnp.float32),
    mesh=_mesh,
    scratch_types=[
        pltpu.VMEM_SHARED((NACC, H), jnp.float32),
        pltpu.VMEM_SHARED((NACC, H), jnp.float32),
        pltpu.VMEM((CPT, CHUNK), jnp.int32),
        pltpu.VMEM((CPT, CHUNK), jnp.int32),
        [pltpu.VMEM((CHUNK, H), jnp.float32) for _ in range(NBUF)],
        [pltpu.SemaphoreType.DMA for _ in range(NBUF)],
        [pltpu.SemaphoreType.DMA for _ in range(NBUF)],
    ],
    compiler_params=_sc_params,
)


def _deg_body(ones_hbm, zeros_hbm, dst_hbm, out_hbm, acc, ones_v, dstb, sem):
    c = lax.axis_index("c")
    s = lax.axis_index("s")
    wid = s * NC + c
    pltpu.sync_copy(dst_hbm.at[wid], dstb)
    pltpu.sync_copy(zeros_hbm.at[pl.ds(s * NPT, NPT)], acc.at[pl.ds(s * NPT, NPT)])
    pltpu.sync_copy(ones_hbm, ones_v)
    plsc.subcore_barrier()

    def fire(k, carry):
        pltpu.async_copy(ones_v, acc.at[dstb.at[k]], sem, add=True)
        return carry

    lax.fori_loop(0, CPT, fire, 0)

    def drain(k, carry):
        pltpu.make_async_copy(ones_v, acc.at[dstb.at[0]], sem).wait()
        return carry

    lax.fori_loop(0, CPT, drain, 0)
    plsc.subcore_barrier()
    pltpu.sync_copy(acc.at[pl.ds(s * NPT, NPT)], out_hbm.at[c, pl.ds(s * NPT, NPT)])


_deg_call = pl.kernel(
    _deg_body,
    out_type=jax.ShapeDtypeStruct((NC, NP_, DEGW), jnp.float32),
    mesh=_mesh,
    scratch_types=[
        pltpu.VMEM_SHARED((NACC, DEGW), jnp.float32),
        pltpu.VMEM((CHUNK, DEGW), jnp.float32),
        pltpu.VMEM((CPT, CHUNK), jnp.int32),
        pltpu.SemaphoreType.DMA,
    ],
    compiler_params=_sc_params,
)


_RB = 1024
NPK = NP_ // 2


def _mm_body(x_ref, w_ref, o_ref):
    o_ref[...] = jnp.dot(x_ref[...], w_ref[...], preferred_element_type=jnp.float32)


def _mm(x, w):
    n, d = x.shape
    h = w.shape[1]
    return pl.pallas_call(
        _mm_body,
        grid=(n // _RB,),
        in_specs=[
            pl.BlockSpec((_RB, d), lambda i: (i, 0)),
            pl.BlockSpec((d, h), lambda i: (0, 0)),
        ],
        out_specs=pl.BlockSpec((_RB, h), lambda i: (i, 0)),
        out_shape=jax.ShapeDtypeStruct((n, h), jnp.float32),
    )(x, w)


def _dinv16_body(degp_ref, o_ref):
    deg = degp_ref[0] + degp_ref[1] + 1.0
    o_ref[...] = lax.rsqrt(deg)


def _dinv16(degpv):
    r = degpv.shape[1]
    return pl.pallas_call(
        _dinv16_body,
        grid=(r // 640,),
        in_specs=[pl.BlockSpec((NC, 640, 128), lambda i: (0, i, 0))],
        out_specs=pl.BlockSpec((640, 128), lambda i: (i, 0)),
        out_shape=jax.ShapeDtypeStruct((r, 128), jnp.float32),
    )(degpv)


def _scale_body(d16_ref, t_ref, u_ref, y_ref, dinv_ref):
    dinvp = jnp.dot(d16_ref[...], t_ref[...], preferred_element_type=jnp.float32)
    y_ref[...] = dinvp * u_ref[...]
    dinv_ref[...] = dinvp


def _scale(d16r, t, u):
    return pl.pallas_call(
        _scale_body,
        grid=(NPK // _RB,),
        in_specs=[
            pl.BlockSpec((_RB, 32), lambda i: (i, 0)),
            pl.BlockSpec((32, 128), lambda i: (0, 0)),
            pl.BlockSpec((_RB, 128), lambda i: (i, 0)),
        ],
        out_specs=[
            pl.BlockSpec((_RB, 128), lambda i: (i, 0)),
            pl.BlockSpec((_RB, 128), lambda i: (i, 0)),
        ],
        out_shape=[
            jax.ShapeDtypeStruct((NPK, 128), jnp.float32),
            jax.ShapeDtypeStruct((NPK, 128), jnp.float32),
        ],
    )(d16r, t, u)


def _next_body(p_ref, yprev_ref, dinv_ref, b_ref, w_ref, y_ref):
    agg = p_ref[0] + p_ref[1] - yprev_ref[...]
    h = jnp.maximum(dinv_ref[...] * agg + b_ref[...], 0.0)
    y_ref[...] = dinv_ref[...] * jnp.dot(
        h, w_ref[...], preferred_element_type=jnp.float32
    )


def _next_layer(pv, yprev, dinv, b2, w2):
    return pl.pallas_call(
        _next_body,
        grid=(NPK // _RB,),
        in_specs=[
            pl.BlockSpec((NC, _RB, 128), lambda i: (0, i, 0)),
            pl.BlockSpec((_RB, 128), lambda i: (i, 0)),
            pl.BlockSpec((_RB, 128), lambda i: (i, 0)),
            pl.BlockSpec((1, 128), lambda i: (0, 0)),
            pl.BlockSpec((128, 128), lambda i: (0, 0)),
        ],
        out_specs=pl.BlockSpec((_RB, 128), lambda i: (i, 0)),
        out_shape=jax.ShapeDtypeStruct((NPK, 128), jnp.float32),
    )(pv, yprev, dinv, b2, w2)


def _final_body(p_ref, yprev_ref, dinv_ref, b_ref, wlin_ref, blin_ref, o_ref):
    agg = p_ref[0] + p_ref[1] - yprev_ref[...]
    h = jnp.maximum(dinv_ref[...] * agg + b_ref[...], 0.0)
    lg = jnp.dot(h, wlin_ref[...], preferred_element_type=jnp.float32)
    lg = lg + blin_ref[...]

    def lsm(l):
        m = jnp.max(l, axis=1, keepdims=True)
        return l - (m + jnp.log(jnp.sum(jnp.exp(l - m), axis=1, keepdims=True)))

    o_ref[...] = jnp.concatenate([lsm(lg[:, 0:2]), lsm(lg[:, 2:4])], axis=1)


_FB = 1000


def _final(pv, yprev, dinv, b2, wlin2, blin2):
    return pl.pallas_call(
        _final_body,
        grid=(N // 2 // _FB,),
        in_specs=[
            pl.BlockSpec((NC, _FB, 128), lambda i: (0, i, 0)),
            pl.BlockSpec((_FB, 128), lambda i: (i, 0)),
            pl.BlockSpec((_FB, 128), lambda i: (i, 0)),
            pl.BlockSpec((1, 128), lambda i: (0, 0)),
            pl.BlockSpec((128, 4), lambda i: (0, 0)),
            pl.BlockSpec((1, 4), lambda i: (0, 0)),
        ],
        out_specs=pl.BlockSpec((_FB, 4), lambda i: (i, 0)),
        out_shape=jax.ShapeDtypeStruct((N // 2, 4), jnp.float32),
    )(pv, yprev, dinv, b2, wlin2, blin2)


def _blockdiag(w):
    d, h = w.shape
    z1 = jnp.zeros((d, h), jnp.float32)
    top = jnp.concatenate([w, z1], axis=1)
    bot = jnp.concatenate([z1, w], axis=1)
    return jnp.concatenate([top, bot], axis=0)


@jax.jit
def kernel(x, edge_index, W1, b1, W2, b2, W3, b3, Wlin, blin):
    srcp = edge_index[0].astype(jnp.int32).reshape(NW, CPT, CHUNK)
    dstp = edge_index[1].astype(jnp.int32).reshape(NW, CPT, CHUNK)

    ones = jnp.ones((CHUNK, DEGW), jnp.float32)
    zeros = jnp.zeros((NP_, DEGW), jnp.float32)
    xp = jnp.pad(x, ((0, NP_ - N), (0, 0)))

    w1b = _blockdiag(W1)
    w2b = _blockdiag(W2)
    w3b = _blockdiag(W3)
    wlb = _blockdiag(Wlin)
    b1x = jnp.concatenate([b1, b1]).reshape(1, 2 * H)
    b2x = jnp.concatenate([b2, b2]).reshape(1, 2 * H)
    b3x = jnp.concatenate([b3, b3]).reshape(1, 2 * H)
    blx = jnp.concatenate([blin, blin]).reshape(1, 2 * C)
    t = jnp.zeros((32, 128), jnp.float32)
    t = t.at[0, 0:64].set(1.0).at[16, 64:128].set(1.0)

    degp = _deg_call(ones, zeros, dstp)
    u1 = _mm(xp.reshape(NPK, 2 * D), w1b)
    d16 = _dinv16(degp.reshape(NC, NP_ * DEGW // 128, 128))
    y1, dinv = _scale(d16.reshape(NPK, 32), t, u1)

    p1 = _agg_call(y1.reshape(NP_, H), srcp, dstp)
    y2 = _next_layer(p1.reshape(NC, NPK, 128), y1, dinv, b1x, w2b)
    p2 = _agg_call(y2.reshape(NP_, H), srcp, dstp)
    y3 = _next_layer(p2.reshape(NC, NPK, 128), y2, dinv, b2x, w3b)
    p3 = _agg_call(y3.reshape(NP_, H), srcp, dstp)
    out = _final(p3.reshape(NC, NPK, 128), y3, dinv, b3x, wlb, blx)
    return out.reshape(N, C)
